# baseline (speedup 1.0000x reference)
import functools

import jax
import jax.numpy as jnp
from jax import lax
from jax.experimental import pallas as pl
from jax.experimental.pallas import tpu as pltpu
from jax.experimental.pallas import tpu_sc as plsc

B = 16384
F = 16
NC, NS = 2, 16
NW = NC * NS
BPW = B // NW
CHUNK = 128
NCHUNK = BPW // CHUNK

@functools.cache
def _make_sc_gather():
    mesh = plsc.VectorSubcoreMesh(
        core_axis_name="c", subcore_axis_name="s", num_cores=NC, num_subcores=NS
    )

    @functools.partial(
        pl.kernel,
        out_type=[jax.ShapeDtypeStruct((B, F), jnp.float32)] * 4,
        mesh=mesh,
        scratch_types=[
            pltpu.VMEM((NCHUNK, CHUNK), jnp.int32),
            pltpu.VMEM((NCHUNK, CHUNK), jnp.int32),
            pltpu.VMEM((BPW, F), jnp.float32),
            pltpu.VMEM((BPW, F), jnp.float32),
            pltpu.VMEM((BPW, F), jnp.float32),
            pltpu.VMEM((BPW, F), jnp.float32),
            pltpu.SemaphoreType.DMA,
        ],
        compiler_params=pltpu.CompilerParams(use_tc_tiling_on_sc=False),
    )
    def sc_gather(uids, iids, gu_t, gi_t, mu_t, mi_t,
                  gu_o, gi_o, mu_o, mi_o,
                  uidx, iidx, gu_v, gi_v, mu_v, mi_v, sem):
        wid = lax.axis_index("s") * NC + lax.axis_index("c")
        base = wid * BPW
        for j in range(NCHUNK):
            pltpu.sync_copy(uids.at[pl.ds(base + j * CHUNK, CHUNK)], uidx.at[j])
            pltpu.sync_copy(iids.at[pl.ds(base + j * CHUNK, CHUNK)], iidx.at[j])
        copies = []
        for table, idx, buf in ((gu_t, uidx, gu_v), (gi_t, iidx, gi_v),
                                (mu_t, uidx, mu_v), (mi_t, iidx, mi_v)):
            for j in range(NCHUNK):
                copies.append(pltpu.async_copy(
                    table.at[idx.at[j]], buf.at[pl.ds(j * CHUNK, CHUNK)], sem))
        for c in copies:
            c.wait()
        pltpu.sync_copy(gu_v, gu_o.at[pl.ds(base, BPW)])
        pltpu.sync_copy(gi_v, gi_o.at[pl.ds(base, BPW)])
        pltpu.sync_copy(mu_v, mu_o.at[pl.ds(base, BPW)])
        pltpu.sync_copy(mi_v, mi_o.at[pl.ds(base, BPW)])

    return sc_gather


BM = 2048


def _tc_head_body(gu, gi, mu, mi, gmf_w, W1, W2, W3, W4, Wf, bvec, out):
    g = gu[...] * gi[...]
    gmf = jnp.dot(g, gmf_w[...], preferred_element_type=jnp.float32)
    gmf = gmf + bvec[0, 0]
    w1 = W1[...]
    h = jnp.dot(mu[...], w1[:F], preferred_element_type=jnp.float32)
    h = h + jnp.dot(mi[...], w1[F:], preferred_element_type=jnp.float32)
    h = jnp.maximum(h + bvec[0, 1:1 + F], 0.0)
    h = jnp.maximum(
        jnp.dot(h, W2[...], preferred_element_type=jnp.float32)
        + bvec[0, 1 + F:1 + 2 * F], 0.0)
    h = jnp.maximum(
        jnp.dot(h, W3[...], preferred_element_type=jnp.float32)
        + bvec[0, 1 + 2 * F:1 + 3 * F], 0.0)
    h = jnp.maximum(
        jnp.dot(h, W4[...], preferred_element_type=jnp.float32)
        + bvec[0, 1 + 3 * F:1 + 4 * F], 0.0)
    mlp = jnp.dot(h, Wf[...], preferred_element_type=jnp.float32)
    mlp = mlp + bvec[0, 1 + 4 * F]
    z = gmf * bvec[0, 2 + 4 * F] + mlp * bvec[0, 3 + 4 * F] + bvec[0, 4 + 4 * F]
    out[...] = 1.0 / (1.0 + jnp.exp(-z))


def _tc_head(gu, gi, mu, mi, gmf_w, W1, W2, W3, W4, Wf, bvec):
    row = lambda i: (i, 0)
    full = lambda i: (0, 0)
    return pl.pallas_call(
        _tc_head_body,
        grid=(B // BM,),
        in_specs=[
            pl.BlockSpec((BM, F), row),
            pl.BlockSpec((BM, F), row),
            pl.BlockSpec((BM, F), row),
            pl.BlockSpec((BM, F), row),
            pl.BlockSpec((F, 1), full),
            pl.BlockSpec((2 * F, F), full),
            pl.BlockSpec((F, F), full),
            pl.BlockSpec((F, F), full),
            pl.BlockSpec((F, F), full),
            pl.BlockSpec((F, 1), full),
            pl.BlockSpec((1, 5 + 4 * F), full),
        ],
        out_specs=pl.BlockSpec((BM, 1), row),
        out_shape=jax.ShapeDtypeStruct((B, 1), jnp.float32),
    )(gu, gi, mu, mi, gmf_w, W1, W2, W3, W4, Wf, bvec)


def kernel(user_ids, item_ids, gmf_user_emb, gmf_item_emb, gmf_w, gmf_b,
           mlp_user_emb, mlp_item_emb, W1, b1, W2, b2, W3, b3, W4, b4,
           Wf, bf, Wo, bo):
    uids = user_ids.astype(jnp.int32)
    iids = item_ids.astype(jnp.int32)
    gu, gi, mu, mi = _make_sc_gather()(
        uids, iids, gmf_user_emb, gmf_item_emb, mlp_user_emb, mlp_item_emb)
    bvec = jnp.concatenate(
        [gmf_b, b1, b2, b3, b4, bf, Wo[0], Wo[1], bo]).reshape(1, 5 + 4 * F)
    return _tc_head(gu, gi, mu, mi, gmf_w, W1, W2, W3, W4, Wf, bvec)


# speedup vs baseline: 5.6211x; 5.6211x over previous
import functools

import jax
import jax.numpy as jnp
from jax import lax
from jax.experimental import pallas as pl
from jax.experimental.pallas import tpu as pltpu
from jax.experimental.pallas import tpu_sc as plsc

B = 16384
F = 16
V = 1_000_000
NC, NS = 2, 16
NW = NC * NS
BPW = B // NW
CHUNK = 128
NCHUNK = BPW // CHUNK

TW = 2048
Q = 4
QTW = Q * TW
STEPS = -(-V // QTW)
WROWS = STEPS * TW
INBLOCKS = -(-V // TW)


def _reformat_body(gu0, mu0, gu1, mu1, gu2, mu2, gu3, mu3,
                   gi0, mi0, gi1, mi1, gi2, mi2, gi3, mi3, ou, oi):
    xu = jnp.concatenate(
        [gu0[...], mu0[...], gu1[...], mu1[...],
         gu2[...], mu2[...], gu3[...], mu3[...]], axis=0)
    ou[...] = xu.T
    xi = jnp.concatenate(
        [gi0[...], mi0[...], gi1[...], mi1[...],
         gi2[...], mi2[...], gi3[...], mi3[...]], axis=0)
    oi[...] = xi.T


def _reformat(guT, muT, giT, miT):
    def spec(q):
        return pl.BlockSpec(
            (F, TW), lambda j, q=q: (0, jnp.minimum(Q * j + q, INBLOCKS - 1)))

    ins, args = [], []
    for g_t, m_t in ((guT, muT), (giT, miT)):
        for q in range(Q):
            ins.extend([spec(q), spec(q)])
            args.extend([g_t, m_t])
    return pl.pallas_call(
        _reformat_body,
        grid=(STEPS,),
        in_specs=ins,
        out_specs=[pl.BlockSpec((TW, 8 * F), lambda j: (j, 0))] * 2,
        out_shape=[jax.ShapeDtypeStruct((WROWS, 8 * F), jnp.float32)] * 2,
    )(*args)


@functools.cache
def _make_sc_gather():
    mesh = plsc.VectorSubcoreMesh(
        core_axis_name="c", subcore_axis_name="s", num_cores=NC, num_subcores=NS
    )

    @functools.partial(
        pl.kernel,
        out_type=[jax.ShapeDtypeStruct((B * 32 // 128, 128), jnp.float32)] * 2,
        mesh=mesh,
        scratch_types=[
            pltpu.VMEM((NCHUNK, CHUNK), jnp.int32),
            pltpu.VMEM((NCHUNK, CHUNK), jnp.int32),
            pltpu.VMEM((NCHUNK, CHUNK), jnp.int32),
            pltpu.VMEM((NCHUNK, CHUNK), jnp.int32),
            pltpu.VMEM((CHUNK, 128), jnp.float32),
            pltpu.VMEM((CHUNK, 128), jnp.float32),
            pltpu.VMEM((CHUNK, 128), jnp.float32),
            pltpu.VMEM((CHUNK, 128), jnp.float32),
            pltpu.VMEM((BPW // 4, 128), jnp.float32),
            pltpu.VMEM((BPW // 4, 128), jnp.float32),
            pltpu.SemaphoreType.DMA,
        ],
        compiler_params=pltpu.CompilerParams(needs_layout_passes=False),
    )
    def sc_gather(uids, iids, wide_u, wide_i, out_u, out_i,
                  urow, uoff, irow, ioff, ubufa, ubufb, ibufa, ibufb,
                  pu, pi, sem):
        wid = lax.axis_index("s") * NC + lax.axis_index("c")
        base = pl.multiple_of(wid * BPW, BPW)
        prow0 = pl.multiple_of(wid * (BPW // 4), BPW // 4)

        for ids, rbuf, obuf in ((uids, urow, uoff), (iids, irow, ioff)):
            for j in range(NCHUNK):
                pltpu.sync_copy(ids.at[pl.ds(base + j * CHUNK, CHUNK)], rbuf.at[j])
            for j in range(NCHUNK):
                for g in range(CHUNK // 16):
                    v = rbuf[j, pl.ds(g * 16, 16)]
                    row = ((v >> 13) << 11) + (v & 2047)
                    off = ((v >> 11) & 3) << 5
                    obuf[j, pl.ds(g * 16, 16)] = off
                    rbuf[j, pl.ds(g * 16, 16)] = row

        ubufs = (ubufa, ubufb)
        ibufs = (ibufa, ibufb)

        def fire(j):
            return (
                pltpu.async_copy(wide_u.at[urow.at[j]], ubufs[j % 2], sem),
                pltpu.async_copy(wide_i.at[irow.at[j]], ibufs[j % 2], sem),
            )

        def extract(j):
            ub, ib = ubufs[j % 2], ibufs[j % 2]

            def group(g, _):
                t = lax.iota(jnp.int32, 16) + g * 16
                s = t + j * CHUNK
                prow = s >> 2
                pcol = (s & 3) << 5
                offu = uoff[j, pl.ds(g * 16, 16)]
                offi = ioff[j, pl.ds(g * 16, 16)]
                for kk in range(32):
                    vu = plsc.load_gather(ub, [t, offu + kk])
                    plsc.store_scatter(pu, [prow, pcol + kk], vu)
                    vi = plsc.load_gather(ib, [t, offi + kk])
                    plsc.store_scatter(pi, [prow, pcol + kk], vi)
                return _

            lax.fori_loop(0, CHUNK // 16, group, None)

        pend = fire(0)
        for j in range(NCHUNK):
            nxt = fire(j + 1) if j + 1 < NCHUNK else None
            for c in pend:
                c.wait()
            extract(j)
            pend = nxt

        pltpu.sync_copy(pu, out_u.at[pl.ds(prow0, BPW // 4)])
        pltpu.sync_copy(pi, out_i.at[pl.ds(prow0, BPW // 4)])

    return sc_gather


PB = 512
NB = 4 * F


def _head_body(up, ip, kgw, k1a, k1b, k2, k3, k4, kf, bvec, out):
    u = up[...]
    it = ip[...]
    g = u * it
    gmf = jnp.dot(g, kgw[...], preferred_element_type=jnp.float32)
    gmf = gmf + bvec[0, 0]
    h = jnp.dot(u, k1a[...], preferred_element_type=jnp.float32)
    h = h + jnp.dot(it, k1b[...], preferred_element_type=jnp.float32)
    h = jnp.maximum(h + bvec[0:1, 1:1 + NB], 0.0)
    h = jnp.maximum(
        jnp.dot(h, k2[...], preferred_element_type=jnp.float32)
        + bvec[0:1, 1 + NB:1 + 2 * NB], 0.0)
    h = jnp.maximum(
        jnp.dot(h, k3[...], preferred_element_type=jnp.float32)
        + bvec[0:1, 1 + 2 * NB:1 + 3 * NB], 0.0)
    h = jnp.maximum(
        jnp.dot(h, k4[...], preferred_element_type=jnp.float32)
        + bvec[0:1, 1 + 3 * NB:1 + 4 * NB], 0.0)
    mlp = jnp.dot(h, kf[...], preferred_element_type=jnp.float32)
    mlp = mlp + bvec[0, 1 + 4 * NB]
    z = gmf * bvec[0, 2 + 4 * NB] + mlp * bvec[0, 3 + 4 * NB] + bvec[0, 4 + 4 * NB]
    out[...] = 1.0 / (1.0 + jnp.exp(-z))


def _head(up, ip, kgw, k1a, k1b, k2, k3, k4, kf, bvec):
    row = lambda i: (i, 0)
    full = lambda i: (0, 0)
    nrows = B * 32 // 128
    return pl.pallas_call(
        _head_body,
        grid=(nrows // PB,),
        in_specs=[
            pl.BlockSpec((PB, 128), row),
            pl.BlockSpec((PB, 128), row),
            pl.BlockSpec((128, 4), full),
            pl.BlockSpec((128, NB), full),
            pl.BlockSpec((128, NB), full),
            pl.BlockSpec((NB, NB), full),
            pl.BlockSpec((NB, NB), full),
            pl.BlockSpec((NB, NB), full),
            pl.BlockSpec((NB, 4), full),
            pl.BlockSpec((1, 5 + 4 * NB), full),
        ],
        out_specs=pl.BlockSpec((PB, 4), row),
        out_shape=jax.ShapeDtypeStruct((nrows, 4), jnp.float32),
    )(up, ip, kgw, k1a, k1b, k2, k3, k4, kf, bvec)


def kernel(user_ids, item_ids, gmf_user_emb, gmf_item_emb, gmf_w, gmf_b,
           mlp_user_emb, mlp_item_emb, W1, b1, W2, b2, W3, b3, W4, b4,
           Wf, bf, Wo, bo):
    uids = user_ids.astype(jnp.int32)
    iids = item_ids.astype(jnp.int32)
    wide_u, wide_i = _reformat(gmf_user_emb.T, mlp_user_emb.T,
                               gmf_item_emb.T, mlp_item_emb.T)
    up, ip_ = _make_sc_gather()(uids, iids, wide_u, wide_i)

    eye4 = jnp.eye(4, dtype=jnp.float32)
    z16 = jnp.zeros((F, F), dtype=jnp.float32)
    z161 = jnp.zeros((F, 1), dtype=jnp.float32)
    kgw = jnp.kron(eye4, jnp.concatenate([gmf_w, z161], axis=0))
    k1a = jnp.kron(eye4, jnp.concatenate([z16, W1[:F]], axis=0))
    k1b = jnp.kron(eye4, jnp.concatenate([z16, W1[F:]], axis=0))
    k2 = jnp.kron(eye4, W2)
    k3 = jnp.kron(eye4, W3)
    k4 = jnp.kron(eye4, W4)
    kf = jnp.kron(eye4, Wf)
    bvec = jnp.concatenate(
        [gmf_b, jnp.tile(b1, 4), jnp.tile(b2, 4), jnp.tile(b3, 4),
         jnp.tile(b4, 4), bf, Wo[0], Wo[1], bo]).reshape(1, 5 + 4 * NB)
    out_p = _head(up, ip_, kgw, k1a, k1b, k2, k3, k4, kf, bvec)
    return out_p.reshape(B, 1)
